# baseline (speedup 1.0000x reference)
import functools

import jax
import jax.numpy as jnp
from jax import lax
from jax.experimental import pallas as pl
from jax.experimental.pallas import tpu as pltpu
from jax.experimental.pallas import tpu_sc as plsc

N_FIELDS = 26
VOCAB = 100000
EMBED = 16
COND_DIM = N_FIELDS * EMBED
BATCH = 16384

NW = 32
ROWS_PER_W = COND_DIM // NW
BCH = 4096
NCH = BATCH // BCH
L = 16
QSTARTS = (0, 25088, 50176, 75264)
QLENS = (25088, 25088, 25088, 24704)
TAIL0 = 99968
TAILW = VOCAB - TAIL0
CPACK = BATCH + EMBED * TAILW


def _make_sc_gather():
    mesh = plsc.VectorSubcoreMesh(core_axis_name="c", subcore_axis_name="s")

    @functools.partial(
        pl.kernel,
        mesh=mesh,
        out_type=jax.ShapeDtypeStruct((COND_DIM, BATCH), jnp.float32),
        scratch_types=[
            pltpu.VMEM((VOCAB,), jnp.float32),
            pltpu.VMEM((BATCH,), jnp.int32),
            pltpu.VMEM((EMBED * TAILW,), jnp.int32),
            pltpu.VMEM((BCH,), jnp.float32),
            pltpu.VMEM((BCH,), jnp.float32),
            pltpu.SemaphoreType.DMA,
            pltpu.SemaphoreType.DMA,
            pltpu.SemaphoreType.DMA,
            pltpu.SemaphoreType.DMA,
            pltpu.SemaphoreType.DMA,
            pltpu.SemaphoreType.DMA,
            pltpu.SemaphoreType.DMA,
        ],
        compiler_params=pltpu.CompilerParams(
            use_tc_tiling_on_sc=True,
            needs_layout_passes=False,
            disable_bounds_checks=True,
        ),
    )
    def gather_k(ttab_hbm, cpack_hbm, xt_hbm, row_v, idx_v, tailb_v, ob0, ob1,
                 sem0, sem1, semr, semq0, semq1, semq2, semq3):
        wid = lax.axis_index("s") * 2 + lax.axis_index("c")
        r0 = wid * ROWS_PER_W

        obufs = (ob0, ob1)
        osems = (sem0, sem1)
        qsems = (semq0, semq1, semq2, semq3)

        def fetch_row(r):
            for q in range(len(QSTARTS)):
                pltpu.async_copy(
                    ttab_hbm.at[r, pl.ds(QSTARTS[q], QLENS[q])],
                    row_v.at[pl.ds(QSTARTS[q], QLENS[q])],
                    qsems[q],
                )

        def wait_row(r):
            for q in range(len(QSTARTS)):
                pltpu.make_async_copy(
                    ttab_hbm.at[r, pl.ds(QSTARTS[q], QLENS[q])],
                    row_v.at[pl.ds(QSTARTS[q], QLENS[q])],
                    qsems[q],
                ).wait()

        def do_row(r, _):
            f = r // EMBED
            @pl.when(jnp.logical_or(r == r0, lax.rem(r, EMBED) == 0))
            def _load_idx():
                pltpu.async_copy(cpack_hbm.at[f, pl.ds(0, BATCH)], idx_v, semr)
                pltpu.sync_copy(
                    cpack_hbm.at[f, pl.ds(BATCH, EMBED * TAILW)], tailb_v
                )
                pltpu.make_async_copy(
                    cpack_hbm.at[f, pl.ds(0, BATCH)], idx_v, semr
                ).wait()

            fetch_row(r)
            wait_row(r)
            e = lax.rem(r, EMBED)
            for h in range(TAILW // L):
                iv = tailb_v[pl.ds(e * TAILW + h * L, L)]
                row_v[pl.ds(TAIL0 + h * L, L)] = lax.bitcast_convert_type(
                    iv, jnp.float32
                )

            for c in range(NCH):
                ob = obufs[c % 2]
                sem = osems[c % 2]
                if c >= 2:
                    pltpu.make_async_copy(ob, xt_hbm.at[r, pl.ds(0, BCH)], sem).wait()

                @plsc.parallel_loop(0, BCH, L, unroll=8)
                def _gather(i):
                    idx = idx_v[pl.ds(c * BCH + i, L)]
                    ob[pl.ds(i, L)] = plsc.load_gather(row_v, [idx])

                pltpu.async_copy(ob, xt_hbm.at[r, pl.ds(c * BCH, BCH)], sem)

            pltpu.make_async_copy(ob0, xt_hbm.at[r, pl.ds(0, BCH)], sem0).wait()
            pltpu.make_async_copy(ob1, xt_hbm.at[r, pl.ds(0, BCH)], sem1).wait()
            return 0

        lax.fori_loop(r0, r0 + ROWS_PER_W, do_row, 0)

    return gather_k


_sc_gather = _make_sc_gather()


def _mlp_body(xt_ref, w1_ref, b1_ref, w2_ref, b2_ref, ot_ref):
    xt = xt_ref[...]
    h = jnp.dot(w1_ref[...], xt, preferred_element_type=jnp.float32) + b1_ref[...]
    h = h * jax.nn.sigmoid(h)
    ot_ref[...] = jnp.dot(w2_ref[...], h, preferred_element_type=jnp.float32) + b2_ref[...]


def _mlp_t(xt, w1, b1, w2, b2):
    bn = 2048
    grid = (BATCH // bn,)
    return pl.pallas_call(
        _mlp_body,
        grid=grid,
        in_specs=[
            pl.BlockSpec((COND_DIM, bn), lambda i: (0, i)),
            pl.BlockSpec((COND_DIM, COND_DIM), lambda i: (0, 0)),
            pl.BlockSpec((COND_DIM, 1), lambda i: (0, 0)),
            pl.BlockSpec((COND_DIM, COND_DIM), lambda i: (0, 0)),
            pl.BlockSpec((COND_DIM, 1), lambda i: (0, 0)),
        ],
        out_specs=pl.BlockSpec((COND_DIM, bn), lambda i: (0, i)),
        out_shape=jax.ShapeDtypeStruct((COND_DIM, BATCH), jnp.float32),
    )(xt, w1, b1, w2, b2)


def kernel(condition, tables, W1, b1, W2, b2):
    ttab = tables.transpose(0, 2, 1).reshape(COND_DIM, VOCAB)
    tail_bits = lax.bitcast_convert_type(
        lax.slice(ttab, (0, TAIL0), (COND_DIM, VOCAB)), jnp.int32
    ).reshape(N_FIELDS, EMBED * TAILW)
    cpack = jnp.concatenate([condition.T, tail_bits], axis=1)
    xt = _sc_gather(ttab, cpack)
    ot = _mlp_t(xt, W1, b1.reshape(COND_DIM, 1), W2, b2.reshape(COND_DIM, 1))
    return ot.T


# speedup vs baseline: 1.0434x; 1.0434x over previous
import functools

import jax
import jax.numpy as jnp
from jax import lax
from jax.experimental import pallas as pl
from jax.experimental.pallas import tpu as pltpu
from jax.experimental.pallas import tpu_sc as plsc

N_FIELDS = 26
VOCAB = 100000
EMBED = 16
COND_DIM = N_FIELDS * EMBED
BATCH = 16384

NW = 32
ROWS_PER_W = COND_DIM // NW
BCH = 4096
NCH = BATCH // BCH
L = 16
QSTARTS = (0, 25088, 50176, 75264, 99968)
QLENS = (25088, 25088, 25088, 24704, 32)


def _make_sc_gather():
    mesh = plsc.VectorSubcoreMesh(core_axis_name="c", subcore_axis_name="s")

    @functools.partial(
        pl.kernel,
        mesh=mesh,
        out_type=jax.ShapeDtypeStruct((COND_DIM, BATCH), jnp.float32),
        scratch_types=[
            pltpu.VMEM((VOCAB,), jnp.float32),
            pltpu.VMEM((BATCH,), jnp.int32),
            pltpu.VMEM((BCH,), jnp.float32),
            pltpu.VMEM((BCH,), jnp.float32),
            pltpu.SemaphoreType.DMA,
            pltpu.SemaphoreType.DMA,
            pltpu.SemaphoreType.DMA,
        ],
        compiler_params=pltpu.CompilerParams(
            use_tc_tiling_on_sc=True,
            needs_layout_passes=False,
            disable_bounds_checks=True,
        ),
    )
    def gather_k(ttab_hbm, condt_hbm, xt_hbm, row_v, idx_v, ob0, ob1, sem0, sem1, semr):
        wid = lax.axis_index("s") * 2 + lax.axis_index("c")
        r0 = wid * ROWS_PER_W

        obufs = (ob0, ob1)
        osems = (sem0, sem1)

        def do_row(r, _):
            f = r // EMBED
            @pl.when(jnp.logical_or(r == r0, lax.rem(r, EMBED) == 0))
            def _load_idx():
                pltpu.sync_copy(condt_hbm.at[f], idx_v)

            pltpu.sync_copy(ttab_hbm.at[r], row_v)

            for c in range(NCH):
                ob = obufs[c % 2]
                sem = osems[c % 2]
                if c >= 2:
                    pltpu.make_async_copy(ob, xt_hbm.at[r, pl.ds(0, BCH)], sem).wait()

                @plsc.parallel_loop(0, BCH, L, unroll=16)
                def _gather(i):
                    idx = idx_v[pl.ds(c * BCH + i, L)]
                    ob[pl.ds(i, L)] = plsc.load_gather(row_v, [idx])

                pltpu.async_copy(ob, xt_hbm.at[r, pl.ds(c * BCH, BCH)], sem)

            pltpu.make_async_copy(ob0, xt_hbm.at[r, pl.ds(0, BCH)], sem0).wait()
            pltpu.make_async_copy(ob1, xt_hbm.at[r, pl.ds(0, BCH)], sem1).wait()
            return 0

        lax.fori_loop(r0, r0 + ROWS_PER_W, do_row, 0)

    return gather_k


_sc_gather = _make_sc_gather()


def _mlp_body(xt_ref, w1_ref, b1_ref, w2_ref, b2_ref, ot_ref):
    xt = xt_ref[...]
    h = jnp.dot(w1_ref[...], xt, preferred_element_type=jnp.float32) + b1_ref[...]
    h = h * jax.nn.sigmoid(h)
    ot_ref[...] = jnp.dot(w2_ref[...], h, preferred_element_type=jnp.float32) + b2_ref[...]


def _mlp_t(xt, w1, b1, w2, b2):
    bn = 2048
    grid = (BATCH // bn,)
    return pl.pallas_call(
        _mlp_body,
        grid=grid,
        in_specs=[
            pl.BlockSpec((COND_DIM, bn), lambda i: (0, i)),
            pl.BlockSpec((COND_DIM, COND_DIM), lambda i: (0, 0)),
            pl.BlockSpec((COND_DIM, 1), lambda i: (0, 0)),
            pl.BlockSpec((COND_DIM, COND_DIM), lambda i: (0, 0)),
            pl.BlockSpec((COND_DIM, 1), lambda i: (0, 0)),
        ],
        out_specs=pl.BlockSpec((COND_DIM, bn), lambda i: (0, i)),
        out_shape=jax.ShapeDtypeStruct((COND_DIM, BATCH), jnp.float32),
    )(xt, w1, b1, w2, b2)


def kernel(condition, tables, W1, b1, W2, b2):
    ttab = tables.transpose(0, 2, 1).reshape(COND_DIM, VOCAB)
    condt = condition.T
    xt = _sc_gather(ttab, condt)
    ot = _mlp_t(xt, W1, b1.reshape(COND_DIM, 1), W2, b2.reshape(COND_DIM, 1))
    return ot.T


# speedup vs baseline: 1.0479x; 1.0043x over previous
import functools

import jax
import jax.numpy as jnp
from jax import lax
from jax.experimental import pallas as pl
from jax.experimental.pallas import tpu as pltpu
from jax.experimental.pallas import tpu_sc as plsc

N_FIELDS = 26
VOCAB = 100000
EMBED = 16
COND_DIM = N_FIELDS * EMBED
BATCH = 16384

NW = 32
ROWS_PER_W = COND_DIM // NW
BCH = 4096
NCH = BATCH // BCH
L = 16
QSTARTS = (0, 25088, 50176, 75264, 99968)
QLENS = (25088, 25088, 25088, 24704, 32)


def _make_sc_gather():
    mesh = plsc.VectorSubcoreMesh(core_axis_name="c", subcore_axis_name="s")

    @functools.partial(
        pl.kernel,
        mesh=mesh,
        out_type=jax.ShapeDtypeStruct((COND_DIM, BATCH), jnp.float32),
        scratch_types=[
            pltpu.VMEM((VOCAB,), jnp.float32),
            pltpu.VMEM((BATCH,), jnp.int32),
            pltpu.VMEM((BCH,), jnp.float32),
            pltpu.VMEM((BCH,), jnp.float32),
            pltpu.SemaphoreType.DMA,
            pltpu.SemaphoreType.DMA,
            pltpu.SemaphoreType.DMA,
        ],
        compiler_params=pltpu.CompilerParams(
            use_tc_tiling_on_sc=True,
            needs_layout_passes=False,
            disable_bounds_checks=True,
        ),
    )
    def gather_k(ttab_hbm, condt_hbm, xt_hbm, row_v, idx_v, ob0, ob1, sem0, sem1, semr):
        wid = lax.axis_index("s") * 2 + lax.axis_index("c")
        r0 = wid * ROWS_PER_W

        obufs = (ob0, ob1)
        osems = (sem0, sem1)

        def do_row(r, _):
            f = r // EMBED
            @pl.when(jnp.logical_or(r == r0, lax.rem(r, EMBED) == 0))
            def _load_idx():
                pltpu.sync_copy(condt_hbm.at[f], idx_v)

            pltpu.sync_copy(ttab_hbm.at[r], row_v)

            for c in range(NCH):
                ob = obufs[c % 2]
                sem = osems[c % 2]
                if c >= 2:
                    pltpu.make_async_copy(ob, xt_hbm.at[r, pl.ds(0, BCH)], sem).wait()

                @plsc.parallel_loop(0, BCH, L, unroll=16)
                def _gather(i):
                    idx = idx_v[pl.ds(c * BCH + i, L)]
                    ob[pl.ds(i, L)] = plsc.load_gather(row_v, [idx])

                pltpu.async_copy(ob, xt_hbm.at[r, pl.ds(c * BCH, BCH)], sem)

            pltpu.make_async_copy(ob0, xt_hbm.at[r, pl.ds(0, BCH)], sem0).wait()
            pltpu.make_async_copy(ob1, xt_hbm.at[r, pl.ds(0, BCH)], sem1).wait()
            return 0

        lax.fori_loop(r0, r0 + ROWS_PER_W, do_row, 0)

    return gather_k


_sc_gather = _make_sc_gather()


def _mlp_body(xt_ref, w1_ref, b1_ref, w2_ref, b2_ref, ot_ref):
    xt = xt_ref[...]
    h = jnp.dot(w1_ref[...], xt, preferred_element_type=jnp.float32) + b1_ref[...]
    h = h * jax.nn.sigmoid(h)
    ot_ref[...] = jnp.dot(w2_ref[...], h, preferred_element_type=jnp.float32) + b2_ref[...]


def _mlp_t(xt, w1, b1, w2, b2):
    bn = 4096
    grid = (BATCH // bn,)
    return pl.pallas_call(
        _mlp_body,
        grid=grid,
        in_specs=[
            pl.BlockSpec((COND_DIM, bn), lambda i: (0, i)),
            pl.BlockSpec((COND_DIM, COND_DIM), lambda i: (0, 0)),
            pl.BlockSpec((COND_DIM, 1), lambda i: (0, 0)),
            pl.BlockSpec((COND_DIM, COND_DIM), lambda i: (0, 0)),
            pl.BlockSpec((COND_DIM, 1), lambda i: (0, 0)),
        ],
        out_specs=pl.BlockSpec((COND_DIM, bn), lambda i: (0, i)),
        out_shape=jax.ShapeDtypeStruct((COND_DIM, BATCH), jnp.float32),
    )(xt, w1, b1, w2, b2)


def kernel(condition, tables, W1, b1, W2, b2):
    ttab = tables.transpose(0, 2, 1).reshape(COND_DIM, VOCAB)
    condt = condition.T
    xt = _sc_gather(ttab, condt)
    ot = _mlp_t(xt, W1, b1.reshape(COND_DIM, 1), W2, b2.reshape(COND_DIM, 1))
    return ot.T
